# baseline (speedup 1.0000x reference)
import functools

import jax
import jax.numpy as jnp
from jax import lax
from jax.experimental import pallas as pl
from jax.experimental.pallas import tpu as pltpu
from jax.experimental.pallas import tpu_sc as plsc

D = 64
D2 = 2 * D
B = 16384
R = 1000
LMBDA = 0.0001

NC = 2
NS = 16
NW = NC * NS
BPW = B // NW
NT = 4

IC = 128
CHUNKS = BPW // IC


@functools.cache
def _sc_gather():
    mesh = plsc.VectorSubcoreMesh(core_axis_name="c", subcore_axis_name="s")
    out_t = [jax.ShapeDtypeStruct((B, D2), jnp.float32)] * NT
    scratch = [
        pltpu.VMEM((CHUNKS, IC), jnp.int32),
        pltpu.VMEM((BPW, D2), jnp.float32),
        pltpu.SemaphoreType.DMA,
    ]

    @functools.partial(pl.kernel, mesh=mesh, out_type=out_t,
                       scratch_types=scratch)
    def k(ph, pt, nh, nt, ent,
          o_ph, o_pt, o_nh, o_nt,
          idx_v, rows, sem):
        wid = lax.axis_index("s") * NC + lax.axis_index("c")
        base = wid * BPW
        wrow = wid * CHUNKS
        pairs = [(ph, o_ph), (pt, o_pt), (nh, o_nh), (nt, o_nt)]
        for idx_hbm, out in pairs:
            pltpu.sync_copy(idx_hbm.at[pl.ds(wrow, CHUNKS)], idx_v)
            handles = [
                pltpu.async_copy(ent.at[idx_v.at[j]],
                                 rows.at[pl.ds(j * IC, IC)], sem)
                for j in range(CHUNKS)
            ]
            for h in handles:
                h.wait()
            pltpu.sync_copy(rows, out.at[pl.ds(base, BPW)])

    return k


def _tc_loss(ph, pt, nh, nt, par, pr_idx, nr_idx, rel):
    BLK = 2048

    def body(ph_ref, pt_ref, nh_ref, nt_ref, par_ref,
             pri_ref, nri_ref, rel_ref, out_ref):
        @pl.when(pl.program_id(0) == 0)
        def _():
            out_ref[0, 0] = 0.0

        relv = rel_ref[...]
        cols = lax.broadcasted_iota(jnp.int32, (BLK, R), 1)

        def pick_rel(idx_ref):
            oh = (cols == idx_ref[...]).astype(jnp.float32)
            return jnp.dot(oh, relv, preferred_element_type=jnp.float32)

        parv = par_ref[...]

        def pick_ent(ref, t):
            two = ref[...]
            sel = parv[:, t:t + 1]
            return jnp.where(sel == 1, two[:, D:], two[:, :D])

        phv = pick_ent(ph_ref, 0)
        ptv = pick_ent(pt_ref, 1)
        nhv = pick_ent(nh_ref, 2)
        ntv = pick_ent(nt_ref, 3)
        prv = pick_rel(pri_ref)
        nrv = pick_rel(nri_ref)
        p = jnp.sum(phv * prv * ptv, axis=-1)
        n = jnp.sum(nhv * nrv * ntv, axis=-1)
        lf = jnp.sum(jax.nn.softplus(-p) + jax.nn.softplus(n))
        rg = jnp.sum(phv * phv + ptv * ptv + prv * prv
                     + nhv * nhv + ntv * ntv + nrv * nrv)
        out_ref[0, 0] += lf + LMBDA * rg

    rspec = pl.BlockSpec((BLK, D2), lambda i: (i, 0))
    pspec = pl.BlockSpec((BLK, NT), lambda i: (i, 0))
    ispec = pl.BlockSpec((BLK, 1), lambda i: (i, 0))
    tspec = pl.BlockSpec((R, D), lambda i: (0, 0))
    out = pl.pallas_call(
        body,
        grid=(B // BLK,),
        in_specs=[rspec] * 4 + [pspec] + [ispec] * 2 + [tspec],
        out_specs=pl.BlockSpec(memory_space=pltpu.SMEM),
        out_shape=jax.ShapeDtypeStruct((1, 1), jnp.float32),
    )(ph, pt, nh, nt, par, pr_idx, nr_idx, rel)
    return out[0, 0]


def kernel(pos_h, pos_t, pos_r, neg_h, neg_t, neg_r,
           ent_embeddings, rel_embeddings):
    eidx = [x.astype(jnp.int32) for x in (pos_h, pos_t, neg_h, neg_t)]
    phys = [(x >> 1).reshape(B // IC, IC) for x in eidx]
    par = jnp.stack([x & 1 for x in eidx], axis=1)
    ent2 = ent_embeddings.reshape(-1, D2)
    ph, pt, nh, nt = _sc_gather()(*phys, ent2)
    pr_idx = pos_r.astype(jnp.int32).reshape(B, 1)
    nr_idx = neg_r.astype(jnp.int32).reshape(B, 1)
    return _tc_loss(ph, pt, nh, nt, par, pr_idx, nr_idx, rel_embeddings)


# speedup vs baseline: 1.7035x; 1.7035x over previous
import functools

import jax
import jax.numpy as jnp
from jax import lax
from jax.experimental import pallas as pl
from jax.experimental.pallas import tpu as pltpu
from jax.experimental.pallas import tpu_sc as plsc

D = 64
B = 16384
R = 1000
LMBDA = 0.0001

NC = 2
NS = 16
NW = NC * NS
BPW = B // NW
NT = 4


@functools.cache
def _sc_gather():
    mesh = plsc.VectorSubcoreMesh(core_axis_name="c", subcore_axis_name="s")
    out_t = [jax.ShapeDtypeStruct((B, D), jnp.float32)] * NT

    HB = BPW // 2
    scratch = [
        pltpu.SMEM((BPW,), jnp.int32),
        pltpu.VMEM_SHARED((B,), jnp.int32),
        pltpu.VMEM((2, HB, D), jnp.float32),
        pltpu.SemaphoreType.DMA,
        pltpu.SemaphoreType.DMA,
    ]

    @functools.partial(pl.kernel, mesh=mesh, out_type=out_t,
                       scratch_types=scratch)
    def k(ph, pt, nh, nt, ent,
          o_ph, o_pt, o_nh, o_nt,
          idx_s, idx_sh, rows, sem0, sem1):
        wid = lax.axis_index("s") * NC + lax.axis_index("c")
        base = wid * BPW
        pairs = [(ph, o_ph), (pt, o_pt), (nh, o_nh), (nt, o_nt)]
        sems = [sem0, sem1]
        stages = [(t, h) for t in range(NT) for h in range(2)]

        def drain(s):
            t, h = stages[s]
            buf = s % 2
            out = pairs[t][1]
            pltpu.make_async_copy(
                ent.at[pl.ds(0, HB)], rows.at[buf], sems[buf]
            ).wait()
            pltpu.sync_copy(rows.at[buf],
                            out.at[pl.ds(base + h * HB, HB)])

        for s, (t, h) in enumerate(stages):
            idx_hbm = pairs[t][0]
            if h == 0:
                pltpu.sync_copy(idx_hbm.at[pl.ds(base, BPW)],
                                idx_sh.at[pl.ds(base, BPW)])
                pltpu.sync_copy(idx_sh.at[pl.ds(base, BPW)], idx_s)
            buf = s % 2

            def fire(i, _, buf=buf, h=h, sem=sems[buf]):
                off = idx_s[h * HB + i]
                pltpu.make_async_copy(
                    ent.at[pl.ds(off, 1)],
                    rows.at[buf].at[pl.ds(i, 1)], sem
                ).start()
                return 0

            lax.fori_loop(0, HB, fire, 0)
            if s > 0:
                drain(s - 1)
        drain(len(stages) - 1)

    return k


def _tc_loss(ph, pt, nh, nt, pr_idx, nr_idx, rel):
    BLK = 2048

    def body(ph_ref, pt_ref, nh_ref, nt_ref, pri_ref, nri_ref, rel_ref,
             out_ref):
        @pl.when(pl.program_id(0) == 0)
        def _():
            out_ref[0, 0] = 0.0

        relv = rel_ref[...]
        cols = lax.broadcasted_iota(jnp.int32, (BLK, R), 1)

        def pick(idx_ref):
            oh = (cols == idx_ref[...]).astype(jnp.float32)
            return jnp.dot(oh, relv, preferred_element_type=jnp.float32)

        phv, ptv = ph_ref[...], pt_ref[...]
        nhv, ntv = nh_ref[...], nt_ref[...]
        prv = pick(pri_ref)
        nrv = pick(nri_ref)
        p = jnp.sum(phv * prv * ptv, axis=-1)
        n = jnp.sum(nhv * nrv * ntv, axis=-1)
        lf = jnp.sum(jax.nn.softplus(-p) + jax.nn.softplus(n))
        rg = jnp.sum(phv * phv + ptv * ptv + prv * prv
                     + nhv * nhv + ntv * ntv + nrv * nrv)
        out_ref[0, 0] += lf + LMBDA * rg

    rspec = pl.BlockSpec((BLK, D), lambda i: (i, 0))
    ispec = pl.BlockSpec((BLK, 1), lambda i: (i, 0))
    tspec = pl.BlockSpec((R, D), lambda i: (0, 0))
    out = pl.pallas_call(
        body,
        grid=(B // BLK,),
        in_specs=[rspec] * 4 + [ispec] * 2 + [tspec],
        out_specs=pl.BlockSpec(memory_space=pltpu.SMEM),
        out_shape=jax.ShapeDtypeStruct((1, 1), jnp.float32),
    )(ph, pt, nh, nt, pr_idx, nr_idx, rel)
    return out[0, 0]


def kernel(pos_h, pos_t, pos_r, neg_h, neg_t, neg_r,
           ent_embeddings, rel_embeddings):
    eidx = [x.astype(jnp.int32) for x in (pos_h, pos_t, neg_h, neg_t)]
    ph, pt, nh, nt = _sc_gather()(*eidx, ent_embeddings)
    pr_idx = pos_r.astype(jnp.int32).reshape(B, 1)
    nr_idx = neg_r.astype(jnp.int32).reshape(B, 1)
    return _tc_loss(ph, pt, nh, nt, pr_idx, nr_idx, rel_embeddings)
